# baseline (speedup 1.0000x reference)
import jax
import jax.numpy as jnp
import numpy as np
from jax.experimental import pallas as pl

NUM_BINS = 15
_EDGES_NP = np.arange(NUM_BINS + 1, dtype=np.float32) * np.float32(
    np.float32(1.0) / np.float32(NUM_BINS)
)
_EDGES_NP[-1] = np.float32(1.0)
_EDGES = tuple(float(x) for x in _EDGES_NP)


def _calib_kernel(pro_ref, lab_ref, cnt_ref, csum_ref, asum_ref):
    step = pl.program_id(0)

    @pl.when(step == 0)
    def _init():
        cnt_ref[...] = jnp.zeros_like(cnt_ref)
        csum_ref[...] = jnp.zeros_like(csum_ref)
        asum_ref[...] = jnp.zeros_like(asum_ref)

    p = pro_ref[...]
    q, s, c = p.shape
    conf = jnp.max(p, axis=2)
    col = jax.lax.broadcasted_iota(jnp.int32, (q, s, c), 2)
    pred = jnp.min(jnp.where(p == conf[:, :, None], col, c), axis=2)
    lab = lab_ref[0]
    corr = (pred == lab).astype(jnp.float32)

    cnt_lt = jnp.zeros((q, s), jnp.int32)
    on_edge = jnp.zeros((q, s), jnp.bool_)
    for e in _EDGES:
        cnt_lt += (conf > e).astype(jnp.int32)
        on_edge |= conf == e
    binid = cnt_lt - 1
    invalid = on_edge | (binid < 0) | (binid >= NUM_BINS)
    binid = jnp.where(invalid, NUM_BINS, binid)

    for b in range(NUM_BINS):
        mf = (binid == b).astype(jnp.float32)
        cnt_ref[b, :] += jnp.sum(mf, axis=0)
        csum_ref[b, :] += jnp.sum(conf * mf, axis=0)
        asum_ref[b, :] += jnp.sum(corr * mf, axis=0)


@jax.jit
def kernel(probas, labels):
    n, c = probas.shape
    s = 64
    groups = n // s
    nb = 125
    bq = groups // nb
    probas3 = probas.reshape(groups, s, c)
    labels3 = labels.reshape(nb, bq, s)

    out_shape = jax.ShapeDtypeStruct((NUM_BINS + 1, s), jnp.float32)
    out_spec = pl.BlockSpec((NUM_BINS + 1, s), lambda i: (0, 0))
    cnt, csum, asum = pl.pallas_call(
        _calib_kernel,
        grid=(nb,),
        in_specs=[
            pl.BlockSpec((bq, s, c), lambda i: (i, 0, 0)),
            pl.BlockSpec((1, bq, s), lambda i: (i, 0, 0)),
        ],
        out_specs=[out_spec, out_spec, out_spec],
        out_shape=[out_shape, out_shape, out_shape],
    )(probas3, labels3)

    counts = jnp.sum(cnt[:NUM_BINS], axis=1)
    denom = jnp.maximum(counts, 1.0)
    empty = counts == 0.0
    confs = jnp.where(empty, jnp.nan, jnp.sum(csum[:NUM_BINS], axis=1) / denom)
    accs = jnp.where(empty, jnp.nan, jnp.sum(asum[:NUM_BINS], axis=1) / denom)
    return confs, accs, counts


# speedup vs baseline: 14.4208x; 14.4208x over previous
import jax
import jax.numpy as jnp
import numpy as np
from jax.experimental import pallas as pl

NUM_BINS = 15
_EDGES_NP = np.arange(NUM_BINS + 1, dtype=np.float32) * np.float32(
    np.float32(1.0) / np.float32(NUM_BINS)
)
_EDGES_NP[-1] = np.float32(1.0)
_EDGES = tuple(float(x) for x in _EDGES_NP)


def _calib_kernel(n_total, pro_ref, lab_ref, cnt_ref, csum_ref, asum_ref):
    step = pl.program_id(0)

    @pl.when(step == 0)
    def _init():
        cnt_ref[...] = jnp.zeros_like(cnt_ref)
        csum_ref[...] = jnp.zeros_like(csum_ref)
        asum_ref[...] = jnp.zeros_like(asum_ref)

    p = pro_ref[...]
    bn, c = p.shape
    g = bn // c
    p3 = p.reshape(g, c, c)
    t = jnp.transpose(p3, (0, 2, 1))
    conf = jnp.max(t, axis=1)
    cls = jax.lax.broadcasted_iota(jnp.int32, (g, c, c), 1)
    pred = jnp.min(jnp.where(t == conf[:, None, :], cls, c), axis=1)
    lab = lab_ref[0]
    corr = (pred == lab).astype(jnp.float32)

    cnt_lt = jnp.zeros((g, c), jnp.int32)
    on_edge = jnp.zeros((g, c), jnp.bool_)
    for e in _EDGES:
        cnt_lt += (conf > e).astype(jnp.int32)
        on_edge |= conf == e
    binid = cnt_lt - 1
    invalid = on_edge | (binid < 0) | (binid >= NUM_BINS)
    binid = jnp.where(invalid, NUM_BINS, binid)

    sid = (
        step * bn
        + jax.lax.broadcasted_iota(jnp.int32, (g, c), 0) * c
        + jax.lax.broadcasted_iota(jnp.int32, (g, c), 1)
    )
    binid = jnp.where(sid < n_total, binid, NUM_BINS)
    conf = jnp.where(binid == NUM_BINS, 0.0, conf)

    for b in range(NUM_BINS):
        mf = (binid == b).astype(jnp.float32)
        cnt_ref[b, :] += jnp.sum(mf, axis=0)
        csum_ref[b, :] += jnp.sum(conf * mf, axis=0)
        asum_ref[b, :] += jnp.sum(corr * mf, axis=0)


@jax.jit
def kernel(probas, labels):
    import functools

    n, c = probas.shape
    bn = 64 * c
    nb = (n + bn - 1) // bn
    n_pad = nb * bn - n
    labels3 = jnp.pad(labels, (0, n_pad)).reshape(nb, bn // c, c)

    out_shape = jax.ShapeDtypeStruct((NUM_BINS + 1, c), jnp.float32)
    out_spec = pl.BlockSpec((NUM_BINS + 1, c), lambda i: (0, 0))
    cnt, csum, asum = pl.pallas_call(
        functools.partial(_calib_kernel, n),
        grid=(nb,),
        in_specs=[
            pl.BlockSpec((bn, c), lambda i: (i, 0)),
            pl.BlockSpec((1, bn // c, c), lambda i: (i, 0, 0)),
        ],
        out_specs=[out_spec, out_spec, out_spec],
        out_shape=[out_shape, out_shape, out_shape],
    )(probas, labels3)

    counts = jnp.sum(cnt[:NUM_BINS], axis=1)
    denom = jnp.maximum(counts, 1.0)
    empty = counts == 0.0
    confs = jnp.where(empty, jnp.nan, jnp.sum(csum[:NUM_BINS], axis=1) / denom)
    accs = jnp.where(empty, jnp.nan, jnp.sum(asum[:NUM_BINS], axis=1) / denom)
    return confs, accs, counts


# speedup vs baseline: 15.2593x; 1.0581x over previous
import functools

import jax
import jax.numpy as jnp
import numpy as np
from jax.experimental import pallas as pl

NUM_BINS = 15
_STEP = float(np.float32(1.0) / np.float32(NUM_BINS))


def _calib_kernel(n_total, pro_ref, lab_ref, pk_ref, csum_ref):
    step = pl.program_id(0)

    @pl.when(step == 0)
    def _init():
        pk_ref[...] = jnp.zeros_like(pk_ref)
        csum_ref[...] = jnp.zeros_like(csum_ref)

    p = pro_ref[...]
    bn, c = p.shape
    g = bn // c
    p3 = p.reshape(g, c, c)
    t = jnp.transpose(p3, (0, 2, 1))
    conf = jnp.max(t, axis=1)
    cls = jax.lax.broadcasted_iota(jnp.int32, (g, c, c), 1)
    pred = jnp.min(jnp.where(t == conf[:, None, :], cls, c), axis=1)
    lab = lab_ref[0]
    packed = jnp.where(pred == lab, jnp.int32(65537), jnp.int32(1))

    d = jnp.float32(_STEP)
    b0 = (conf * 15.0).astype(jnp.int32)
    up = jnp.where(b0 >= NUM_BINS - 1, 1.0, (b0.astype(jnp.float32) + 1.0) * d)
    b1 = b0 + (conf >= up).astype(jnp.int32)
    b1f = b1.astype(jnp.float32)
    b2 = b1 - (conf <= b1f * d).astype(jnp.int32)
    b2f = b2.astype(jnp.float32)
    e_lo = b2f * d
    e_hi = jnp.where(b2 >= NUM_BINS - 1, 1.0, (b2f + 1.0) * d)
    valid = (b2 >= 0) & (b2 < NUM_BINS) & (e_lo < conf) & (conf < e_hi)

    sid = (
        step * bn
        + jax.lax.broadcasted_iota(jnp.int32, (g, c), 0) * c
        + jax.lax.broadcasted_iota(jnp.int32, (g, c), 1)
    )
    binid = jnp.where(valid & (sid < n_total), b2, NUM_BINS)
    zero_i = jnp.zeros_like(packed)
    zero_f = jnp.zeros_like(conf)

    for b in range(NUM_BINS):
        m = binid == b
        pk_ref[b, :] += jnp.sum(jnp.where(m, packed, zero_i), axis=0)
        csum_ref[b, :] += jnp.sum(jnp.where(m, conf, zero_f), axis=0)


@jax.jit
def kernel(probas, labels):
    n, c = probas.shape
    bn = 128 * c
    nb = (n + bn - 1) // bn
    n_pad = nb * bn - n
    labels3 = jnp.pad(labels, (0, n_pad)).reshape(nb, bn // c, c)

    out_spec = pl.BlockSpec((NUM_BINS + 1, c), lambda i: (0, 0))
    pk, csum = pl.pallas_call(
        functools.partial(_calib_kernel, n),
        grid=(nb,),
        in_specs=[
            pl.BlockSpec((bn, c), lambda i: (i, 0)),
            pl.BlockSpec((1, bn // c, c), lambda i: (i, 0, 0)),
        ],
        out_specs=[out_spec, out_spec],
        out_shape=[
            jax.ShapeDtypeStruct((NUM_BINS + 1, c), jnp.int32),
            jax.ShapeDtypeStruct((NUM_BINS + 1, c), jnp.float32),
        ],
    )(probas, labels3)

    counts = jnp.sum(pk[:NUM_BINS] & 0xFFFF, axis=1).astype(jnp.float32)
    corrs = jnp.sum(pk[:NUM_BINS] >> 16, axis=1).astype(jnp.float32)
    denom = jnp.maximum(counts, 1.0)
    empty = counts == 0.0
    confs = jnp.where(empty, jnp.nan, jnp.sum(csum[:NUM_BINS], axis=1) / denom)
    accs = jnp.where(empty, jnp.nan, corrs / denom)
    return confs, accs, counts
